# baseline (speedup 1.0000x reference)
import functools

import jax
import jax.numpy as jnp
from jax import lax
from jax.experimental import pallas as pl
from jax.experimental.pallas import tpu as pltpu
from jax.experimental.pallas import tpu_sc as plsc

N_NODES = 10000
N_EDGES = 320000
D = 128
DH = D // 2
BN_EPS = 1e-5

NC = 2
NS = 16
NW = NC * NS
CH = 128
STEPS = 80
IDXB = 16
NIB = STEPS // IDXB
E_PER_W = STEPS * CH
E_PAD = NW * E_PER_W
R_PAD = 10240
RPW = R_PAD // NS
XPW = 632


def _sc_body(x_hbm, x0_hbm, x1_hbm, sidx_hbm, didx_hbm, zrow_hbm, zcnt_hbm,
             part_hbm, cnt_hbm,
             sA_v, dA_v, sB_v, dB_v, rows0_v, rows1_v, rows2_v, rows3_v,
             ones_v, xs_s, acc_s, cnt_s,
             g0, g1, g2, g3, s0, s1, s2, s3, csem, isemA, isemB):
    cid = lax.axis_index("c")
    sid = lax.axis_index("s")
    wid = cid * NS + sid
    rows = (rows0_v, rows1_v, rows2_v, rows3_v)
    gsem = (g0, g1, g2, g3)
    ssem = (s0, s1, s2, s3)

    def idx_start(n, sbuf, dbuf, isem):
        pltpu.async_copy(sidx_hbm.at[wid, n], sbuf, isem)
        pltpu.async_copy(didx_hbm.at[wid, n], dbuf, isem)

    def idx_wait(n, sbuf, dbuf, isem):
        pltpu.make_async_copy(sidx_hbm.at[wid, n], sbuf, isem).wait()
        pltpu.make_async_copy(didx_hbm.at[wid, n], dbuf, isem).wait()

    for i in range(CH // 16):
        ones_v[pl.ds(i * 16, 16)] = jnp.full((16,), 1.0, jnp.float32)

    def gat_start(src, sbuf, j, buf, sem):
        pltpu.async_copy(src.at[sbuf.at[j]], buf, sem)

    def gat_wait(src, sbuf, j, buf, sem):
        pltpu.make_async_copy(src.at[sbuf.at[j]], buf, sem).wait()

    def scat_start(dbuf, j, buf, sem, with_counts):
        pltpu.async_copy(buf, acc_s.at[dbuf.at[j]], sem, add=True)
        if with_counts:
            pltpu.async_copy(ones_v, cnt_s.at[dbuf.at[j]], csem, add=True)

    def scat_wait(dbuf, j, buf, sem):
        pltpu.make_async_copy(buf, acc_s.at[dbuf.at[j]], sem).wait()

    def process_block(xh_hbm, n, sbuf, dbuf, isem, with_counts):
        def src_for(m):
            return xh_hbm if m % 4 == 3 else xs_s

        idx_wait(n, sbuf, dbuf, isem)
        gat_start(src_for(0), sbuf, 0, rows[0], gsem[0])

        def quad(k, carry):
            for m in range(4):
                i = 4 * k + m

                @pl.when(i >= 3)
                def _(i=i, m=m):
                    scat_wait(dbuf, i - 3, rows[(m + 1) % 4], ssem[(m + 1) % 4])

                @pl.when(i + 1 < IDXB)
                def _(i=i, m=m):
                    gat_start(src_for(m + 1), sbuf, i + 1,
                              rows[(m + 1) % 4], gsem[(m + 1) % 4])

                gat_wait(src_for(m), sbuf, i, rows[m], gsem[m])
                scat_start(dbuf, i, rows[m], ssem[m], with_counts)
            return carry

        lax.fori_loop(0, IDXB // 4, quad, 0)

        for j in range(IDXB - 3, IDXB):
            scat_wait(dbuf, j, rows[j % 4], ssem[j % 4])
        if with_counts:
            def cdrain(_, carry):
                pltpu.make_async_copy(ones_v, cnt_s.at[dbuf.at[0]], csem).wait()
                return carry
            lax.fori_loop(0, IDXB, cdrain, 0)

        if n + 2 < NIB:
            idx_start(n + 2, sbuf, dbuf, isem)

    def run_pass(h, xh_hbm, with_counts):
        @pl.when(sid < NS - 1)
        def _():
            pltpu.async_copy(x_hbm.at[pl.ds(sid * XPW, XPW), pl.ds(h * DH, DH)],
                             xs_s.at[pl.ds(sid * XPW, XPW)], g0)

        @pl.when(sid == NS - 1)
        def _():
            pltpu.async_copy(
                x_hbm.at[pl.ds((NS - 1) * XPW, N_NODES - (NS - 1) * XPW),
                         pl.ds(h * DH, DH)],
                xs_s.at[pl.ds((NS - 1) * XPW, N_NODES - (NS - 1) * XPW)], g0)

        idx_start(0, sA_v, dA_v, isemA)
        idx_start(1, sB_v, dB_v, isemB)
        pltpu.sync_copy(zrow_hbm, acc_s.at[pl.ds(sid * RPW, RPW)])
        if with_counts:
            pltpu.sync_copy(zcnt_hbm, cnt_s.at[pl.ds(sid * RPW, RPW)])

        @pl.when(sid < NS - 1)
        def _():
            pltpu.make_async_copy(
                x_hbm.at[pl.ds(sid * XPW, XPW), pl.ds(h * DH, DH)],
                xs_s.at[pl.ds(sid * XPW, XPW)], g0).wait()

        @pl.when(sid == NS - 1)
        def _():
            pltpu.make_async_copy(
                x_hbm.at[pl.ds((NS - 1) * XPW, N_NODES - (NS - 1) * XPW),
                         pl.ds(h * DH, DH)],
                xs_s.at[pl.ds((NS - 1) * XPW, N_NODES - (NS - 1) * XPW)], g0).wait()

        plsc.subcore_barrier()

        for n in range(NIB):
            if n % 2 == 0:
                process_block(xh_hbm, n, sA_v, dA_v, isemA, with_counts)
            else:
                process_block(xh_hbm, n, sB_v, dB_v, isemB, with_counts)
        plsc.subcore_barrier()

        pltpu.sync_copy(acc_s.at[pl.ds(sid * RPW, RPW)],
                        part_hbm.at[h, cid, pl.ds(sid * RPW, RPW)])
        if with_counts:
            pltpu.sync_copy(cnt_s.at[pl.ds(sid * RPW, RPW)],
                            cnt_hbm.at[cid, pl.ds(sid * RPW, RPW)])

    run_pass(0, x0_hbm, True)
    plsc.subcore_barrier()
    run_pass(1, x1_hbm, False)


_sc_agg = pl.kernel(
    _sc_body,
    out_type=[
        jax.ShapeDtypeStruct((2, NC, R_PAD, DH), jnp.float32),
        jax.ShapeDtypeStruct((NC, R_PAD), jnp.float32),
    ],
    mesh=plsc.VectorSubcoreMesh(core_axis_name="c", subcore_axis_name="s"),
    compiler_params=pltpu.CompilerParams(use_tc_tiling_on_sc=False),
    scratch_types=[
        pltpu.VMEM((IDXB, CH), jnp.int32),
        pltpu.VMEM((IDXB, CH), jnp.int32),
        pltpu.VMEM((IDXB, CH), jnp.int32),
        pltpu.VMEM((IDXB, CH), jnp.int32),
        pltpu.VMEM((CH, DH), jnp.float32),
        pltpu.VMEM((CH, DH), jnp.float32),
        pltpu.VMEM((CH, DH), jnp.float32),
        pltpu.VMEM((CH, DH), jnp.float32),
        pltpu.VMEM((CH,), jnp.float32),
        pltpu.VMEM_SHARED((N_NODES, DH), jnp.float32),
        pltpu.VMEM_SHARED((R_PAD, DH), jnp.float32),
        pltpu.VMEM_SHARED((R_PAD,), jnp.float32),
        pltpu.SemaphoreType.DMA,
        pltpu.SemaphoreType.DMA,
        pltpu.SemaphoreType.DMA,
        pltpu.SemaphoreType.DMA,
        pltpu.SemaphoreType.DMA,
        pltpu.SemaphoreType.DMA,
        pltpu.SemaphoreType.DMA,
        pltpu.SemaphoreType.DMA,
        pltpu.SemaphoreType.DMA,
        pltpu.SemaphoreType.DMA,
        pltpu.SemaphoreType.DMA,
    ],
)

BM = 1000
NB = N_NODES // BM


def _tc_body(part_ref, cnt_ref, w_ref, b_ref, g_ref, bt_ref, out_ref,
             pre_vmem, stat_vmem):
    i = pl.program_id(0)

    @pl.when(i < NB)
    def _():
        c = cnt_ref[:, 0:1] + cnt_ref[:, 1:2]
        inv = 1.0 / jnp.maximum(c, 1.0)
        has = jnp.where(c > 0.0, 1.0, 0.0)
        agg0 = (part_ref[0, 0] + part_ref[0, 1]) * inv
        agg1 = (part_ref[1, 0] + part_ref[1, 1]) * inv
        pre = lax.dot_general(agg0, w_ref[:, 0:DH], (((1,), (1,)), ((), ())),
                              preferred_element_type=jnp.float32)
        pre += lax.dot_general(agg1, w_ref[:, DH:D], (((1,), (1,)), ((), ())),
                               preferred_element_type=jnp.float32)
        pre = pre + has * b_ref[...]
        pre_vmem[pl.ds(i * BM, BM), :] = pre

        @pl.when(i == 0)
        def _():
            stat_vmem[...] = jnp.zeros_like(stat_vmem)

        stat_vmem[0:1] += jnp.sum(pre, axis=0, keepdims=True)
        stat_vmem[1:2] += jnp.sum(pre * pre, axis=0, keepdims=True)

    @pl.when(i >= NB)
    def _():
        inv_n = 1.0 / N_NODES
        mu = stat_vmem[0:1] * inv_n
        var = stat_vmem[1:2] * inv_n - mu * mu
        scale = g_ref[...] * lax.rsqrt(var + BN_EPS)
        pre = pre_vmem[pl.ds((i - NB) * BM, BM), :]
        out_ref[...] = jnp.maximum((pre - mu) * scale + bt_ref[...], 0.0)


_tc_fused = pl.pallas_call(
    _tc_body,
    grid=(2 * NB,),
    in_specs=[
        pl.BlockSpec((2, NC, BM, DH), lambda i: (0, 0, jnp.minimum(i, NB - 1), 0)),
        pl.BlockSpec((BM, 2), lambda i: (jnp.minimum(i, NB - 1), 0)),
        pl.BlockSpec((D, D), lambda i: (0, 0)),
        pl.BlockSpec((1, D), lambda i: (0, 0)),
        pl.BlockSpec((1, D), lambda i: (0, 0)),
        pl.BlockSpec((1, D), lambda i: (0, 0)),
    ],
    out_specs=pl.BlockSpec((BM, D), lambda i: (jnp.maximum(i - NB, 0), 0)),
    out_shape=jax.ShapeDtypeStruct((N_NODES, D), jnp.float32),
    scratch_shapes=[
        pltpu.VMEM((N_NODES, D), jnp.float32),
        pltpu.VMEM((2, D), jnp.float32),
    ],
)


def kernel(x, edge_index, W, b, gamma, beta):
    ei = edge_index.astype(jnp.int32)
    pad = E_PAD - N_EDGES
    src = jnp.concatenate([ei[0], jnp.zeros((pad,), jnp.int32)])
    dst = jnp.concatenate([ei[1], jnp.full((pad,), R_PAD - 1, jnp.int32)])
    src = src.reshape(NW, NIB, IDXB, CH)
    dst = dst.reshape(NW, NIB, IDXB, CH)
    zrow = jnp.zeros((RPW, DH), jnp.float32)
    zcnt = jnp.zeros((RPW,), jnp.float32)

    part, cnt = _sc_agg(x, x[:, :DH], x[:, DH:], src, dst, zrow, zcnt)

    return _tc_fused(part, cnt.T, W, b.reshape(1, D),
                     gamma.reshape(1, D), beta.reshape(1, D))


# speedup vs baseline: 1.4113x; 1.4113x over previous
import functools

import jax
import jax.numpy as jnp
from jax import lax
from jax.experimental import pallas as pl
from jax.experimental.pallas import tpu as pltpu
from jax.experimental.pallas import tpu_sc as plsc

N_NODES = 10000
N_EDGES = 320000
D = 128
DH = D // 2
BN_EPS = 1e-5

NC = 2
NS = 16
NW = NC * NS
CH = 128
STEPS = 80
IDXB = 16
NIB = STEPS // IDXB
E_PER_W = STEPS * CH
E_PAD = NW * E_PER_W
R_PAD = 10240
RPW = R_PAD // NS
XPW = 632


def _sc_body(x_hbm, sidx_hbm, didx_hbm, zrow_hbm, zcnt_hbm,
             part_hbm, cnt_hbm,
             sA_v, dA_v, sB_v, dB_v, rows0_v, rows1_v, rows2_v, rows3_v,
             ones_v, xs_s, acc_s, cnt_s,
             g0, g1, g2, g3, s0, s1, s2, s3, csem, isemA, isemB):
    cid = lax.axis_index("c")
    sid = lax.axis_index("s")
    wid = cid * NS + sid
    rows = (rows0_v, rows1_v, rows2_v, rows3_v)
    gsem = (g0, g1, g2, g3)
    ssem = (s0, s1, s2, s3)

    def idx_start(n, sbuf, dbuf, isem):
        pltpu.async_copy(sidx_hbm.at[wid, n], sbuf, isem)
        pltpu.async_copy(didx_hbm.at[wid, n], dbuf, isem)

    def idx_wait(n, sbuf, dbuf, isem):
        pltpu.make_async_copy(sidx_hbm.at[wid, n], sbuf, isem).wait()
        pltpu.make_async_copy(didx_hbm.at[wid, n], dbuf, isem).wait()

    for i in range(CH // 16):
        ones_v[pl.ds(i * 16, 16)] = jnp.full((16,), 1.0, jnp.float32)

    def gat_start(sbuf, j, buf, sem):
        pltpu.async_copy(xs_s.at[sbuf.at[j]], buf, sem)

    def gat_wait(sbuf, j, buf, sem):
        pltpu.make_async_copy(xs_s.at[sbuf.at[j]], buf, sem).wait()

    def scat_start(dbuf, j, buf, sem, with_counts):
        pltpu.async_copy(buf, acc_s.at[dbuf.at[j]], sem, add=True)
        if with_counts:
            pltpu.async_copy(ones_v, cnt_s.at[dbuf.at[j]], csem, add=True)

    def scat_wait(dbuf, j, buf, sem):
        pltpu.make_async_copy(buf, acc_s.at[dbuf.at[j]], sem).wait()

    def process_block(n, sbuf, dbuf, isem, with_counts):
        idx_wait(n, sbuf, dbuf, isem)
        gat_start(sbuf, 0, rows[0], gsem[0])

        def quad(k, carry):
            for m in range(4):
                i = 4 * k + m

                @pl.when(i >= 3)
                def _(i=i, m=m):
                    scat_wait(dbuf, i - 3, rows[(m + 1) % 4], ssem[(m + 1) % 4])

                @pl.when(i + 1 < IDXB)
                def _(i=i, m=m):
                    gat_start(sbuf, i + 1, rows[(m + 1) % 4], gsem[(m + 1) % 4])

                gat_wait(sbuf, i, rows[m], gsem[m])
                scat_start(dbuf, i, rows[m], ssem[m], with_counts)
            return carry

        lax.fori_loop(0, IDXB // 4, quad, 0)

        for j in range(IDXB - 3, IDXB):
            scat_wait(dbuf, j, rows[j % 4], ssem[j % 4])
        if with_counts:
            def cdrain(_, carry):
                pltpu.make_async_copy(ones_v, cnt_s.at[dbuf.at[0]], csem).wait()
                return carry
            lax.fori_loop(0, IDXB, cdrain, 0)

        if n + 2 < NIB:
            idx_start(n + 2, sbuf, dbuf, isem)

    def run_pass(h, with_counts):
        @pl.when(sid < NS - 1)
        def _():
            pltpu.async_copy(x_hbm.at[pl.ds(sid * XPW, XPW), pl.ds(h * DH, DH)],
                             xs_s.at[pl.ds(sid * XPW, XPW)], g0)

        @pl.when(sid == NS - 1)
        def _():
            pltpu.async_copy(
                x_hbm.at[pl.ds((NS - 1) * XPW, N_NODES - (NS - 1) * XPW),
                         pl.ds(h * DH, DH)],
                xs_s.at[pl.ds((NS - 1) * XPW, N_NODES - (NS - 1) * XPW)], g0)

        idx_start(0, sA_v, dA_v, isemA)
        idx_start(1, sB_v, dB_v, isemB)
        pltpu.sync_copy(zrow_hbm, acc_s.at[pl.ds(sid * RPW, RPW)])
        if with_counts:
            pltpu.sync_copy(zcnt_hbm, cnt_s.at[pl.ds(sid * RPW, RPW)])

        @pl.when(sid < NS - 1)
        def _():
            pltpu.make_async_copy(
                x_hbm.at[pl.ds(sid * XPW, XPW), pl.ds(h * DH, DH)],
                xs_s.at[pl.ds(sid * XPW, XPW)], g0).wait()

        @pl.when(sid == NS - 1)
        def _():
            pltpu.make_async_copy(
                x_hbm.at[pl.ds((NS - 1) * XPW, N_NODES - (NS - 1) * XPW),
                         pl.ds(h * DH, DH)],
                xs_s.at[pl.ds((NS - 1) * XPW, N_NODES - (NS - 1) * XPW)], g0).wait()

        plsc.subcore_barrier()

        for n in range(NIB):
            if n % 2 == 0:
                process_block(n, sA_v, dA_v, isemA, with_counts)
            else:
                process_block(n, sB_v, dB_v, isemB, with_counts)
        plsc.subcore_barrier()

        pltpu.sync_copy(acc_s.at[pl.ds(sid * RPW, RPW)],
                        part_hbm.at[h, cid, pl.ds(sid * RPW, RPW)])
        if with_counts:
            pltpu.sync_copy(cnt_s.at[pl.ds(sid * RPW, RPW)],
                            cnt_hbm.at[cid, pl.ds(sid * RPW, RPW)])

    run_pass(0, True)
    plsc.subcore_barrier()
    run_pass(1, False)


_sc_agg = pl.kernel(
    _sc_body,
    out_type=[
        jax.ShapeDtypeStruct((2, NC, R_PAD, DH), jnp.float32),
        jax.ShapeDtypeStruct((NC, R_PAD), jnp.float32),
    ],
    mesh=plsc.VectorSubcoreMesh(core_axis_name="c", subcore_axis_name="s"),
    compiler_params=pltpu.CompilerParams(use_tc_tiling_on_sc=False),
    scratch_types=[
        pltpu.VMEM((IDXB, CH), jnp.int32),
        pltpu.VMEM((IDXB, CH), jnp.int32),
        pltpu.VMEM((IDXB, CH), jnp.int32),
        pltpu.VMEM((IDXB, CH), jnp.int32),
        pltpu.VMEM((CH, DH), jnp.float32),
        pltpu.VMEM((CH, DH), jnp.float32),
        pltpu.VMEM((CH, DH), jnp.float32),
        pltpu.VMEM((CH, DH), jnp.float32),
        pltpu.VMEM((CH,), jnp.float32),
        pltpu.VMEM_SHARED((N_NODES, DH), jnp.float32),
        pltpu.VMEM_SHARED((R_PAD, DH), jnp.float32),
        pltpu.VMEM_SHARED((R_PAD,), jnp.float32),
        pltpu.SemaphoreType.DMA,
        pltpu.SemaphoreType.DMA,
        pltpu.SemaphoreType.DMA,
        pltpu.SemaphoreType.DMA,
        pltpu.SemaphoreType.DMA,
        pltpu.SemaphoreType.DMA,
        pltpu.SemaphoreType.DMA,
        pltpu.SemaphoreType.DMA,
        pltpu.SemaphoreType.DMA,
        pltpu.SemaphoreType.DMA,
        pltpu.SemaphoreType.DMA,
    ],
)

BM = 1000
NB = N_NODES // (2 * BM)
D2 = 2 * D


def _tc_body(part_ref, inv_ref, has_ref, m0_ref, m1_ref, b_ref, g_ref, bt_ref,
             out_ref, pre_vmem, stat_vmem):
    i = pl.program_id(0)

    @pl.when(i < NB)
    def _():
        sc0 = (part_ref[0, 0] + part_ref[0, 1]) * inv_ref[...]
        sc1 = (part_ref[1, 0] + part_ref[1, 1]) * inv_ref[...]
        pre = lax.dot_general(sc0, m0_ref[...], (((1,), (0,)), ((), ())),
                              preferred_element_type=jnp.float32)
        pre += lax.dot_general(sc1, m1_ref[...], (((1,), (0,)), ((), ())),
                               preferred_element_type=jnp.float32)
        hb = has_ref[...]
        pre += jnp.concatenate([hb[:, 0:1] * b_ref[...],
                                hb[:, 1:2] * b_ref[...]], axis=1)
        pre_vmem[pl.ds(i * BM, BM), :] = pre

        @pl.when(i == 0)
        def _():
            stat_vmem[...] = jnp.zeros_like(stat_vmem)

        stat_vmem[0:1] += jnp.sum(pre, axis=0, keepdims=True)
        stat_vmem[1:2] += jnp.sum(pre * pre, axis=0, keepdims=True)

    @pl.when(i >= NB)
    def _():
        inv_n = 1.0 / N_NODES
        mu = (stat_vmem[0:1, 0:D] + stat_vmem[0:1, D:D2]) * inv_n
        ex2 = (stat_vmem[1:2, 0:D] + stat_vmem[1:2, D:D2]) * inv_n
        var = ex2 - mu * mu
        scale = g_ref[...] * lax.rsqrt(var + BN_EPS)
        mu2 = jnp.concatenate([mu, mu], axis=1)
        scale2 = jnp.concatenate([scale, scale], axis=1)
        bt2 = jnp.concatenate([bt_ref[...], bt_ref[...]], axis=1)
        pre = pre_vmem[pl.ds((i - NB) * BM, BM), :]
        out_ref[...] = jnp.maximum((pre - mu2) * scale2 + bt2, 0.0)


_tc_fused = pl.pallas_call(
    _tc_body,
    grid=(2 * NB,),
    in_specs=[
        pl.BlockSpec((2, NC, BM, D), lambda i: (0, 0, jnp.minimum(i, NB - 1), 0)),
        pl.BlockSpec((BM, D), lambda i: (jnp.minimum(i, NB - 1), 0)),
        pl.BlockSpec((BM, 2), lambda i: (jnp.minimum(i, NB - 1), 0)),
        pl.BlockSpec((D, D2), lambda i: (0, 0)),
        pl.BlockSpec((D, D2), lambda i: (0, 0)),
        pl.BlockSpec((1, D), lambda i: (0, 0)),
        pl.BlockSpec((1, D), lambda i: (0, 0)),
        pl.BlockSpec((1, D), lambda i: (0, 0)),
    ],
    out_specs=pl.BlockSpec((BM, D2), lambda i: (jnp.maximum(i - NB, 0), 0)),
    out_shape=jax.ShapeDtypeStruct((N_NODES // 2, D2), jnp.float32),
    scratch_shapes=[
        pltpu.VMEM((N_NODES // 2, D2), jnp.float32),
        pltpu.VMEM((2, D2), jnp.float32),
    ],
)


def kernel(x, edge_index, W, b, gamma, beta):
    ei = edge_index.astype(jnp.int32)
    pad = E_PAD - N_EDGES
    src = jnp.concatenate([ei[0], jnp.zeros((pad,), jnp.int32)])
    dst = jnp.concatenate([ei[1], jnp.full((pad,), R_PAD - 1, jnp.int32)])
    src = src.reshape(NW, NIB, IDXB, CH)
    dst = dst.reshape(NW, NIB, IDXB, CH)
    zrow = jnp.zeros((RPW, DH), jnp.float32)
    zcnt = jnp.zeros((RPW,), jnp.float32)

    part, cnt = _sc_agg(x, src, dst, zrow, zcnt)

    part2 = part.reshape(2, NC, R_PAD // 2, D)
    c = cnt[0] + cnt[1]
    inv = 1.0 / jnp.clip(c, 1.0, None)
    inv2 = jnp.broadcast_to(inv.reshape(R_PAD // 2, 2, 1),
                            (R_PAD // 2, 2, DH)).reshape(R_PAD // 2, D)
    has2 = (c > 0).astype(jnp.float32).reshape(R_PAD // 2, 2)
    w0t = W[:, :DH].T
    w1t = W[:, DH:].T
    z = jnp.zeros((DH, D), jnp.float32)
    m0 = jnp.concatenate([jnp.concatenate([w0t, z], 1),
                          jnp.concatenate([z, w0t], 1)], 0)
    m1 = jnp.concatenate([jnp.concatenate([w1t, z], 1),
                          jnp.concatenate([z, w1t], 1)], 0)

    out2 = _tc_fused(part2, inv2, has2, m0, m1, b.reshape(1, D),
                     gamma.reshape(1, D), beta.reshape(1, D))
    return out2.reshape(N_NODES, D)


# speedup vs baseline: 1.4802x; 1.0489x over previous
import functools

import jax
import jax.numpy as jnp
from jax import lax
from jax.experimental import pallas as pl
from jax.experimental.pallas import tpu as pltpu
from jax.experimental.pallas import tpu_sc as plsc

N_NODES = 10000
N_EDGES = 320000
D = 128
DH = D // 2
BN_EPS = 1e-5

NC = 2
NS = 16
NW = NC * NS
CH = 128
STEPS = 80
IDXB = 16
NIB = STEPS // IDXB
E_PER_W = STEPS * CH
E_PAD = NW * E_PER_W
R_PAD = 10240
RPW = R_PAD // NS
XPW = 632
ZR = 64


def _sc_body(x_hbm, sidx_hbm, didx_hbm,
             part_hbm, cnt_hbm,
             sA_v, dA_v, sB_v, dB_v, rows0_v, rows1_v, rows2_v, rows3_v,
             ones_v, zrow_v, zcnt_v, xs_s, acc_s, cnt_s,
             g0, g1, g2, g3, s0, s1, s2, s3, csem, isemA, isemB):
    cid = lax.axis_index("c")
    sid = lax.axis_index("s")
    wid = cid * NS + sid
    rows = (rows0_v, rows1_v, rows2_v, rows3_v)
    gsem = (g0, g1, g2, g3)
    ssem = (s0, s1, s2, s3)

    def idx_start(n, sbuf, dbuf, isem):
        pltpu.async_copy(sidx_hbm.at[wid, n], sbuf, isem)
        pltpu.async_copy(didx_hbm.at[wid, n], dbuf, isem)

    def idx_wait(n, sbuf, dbuf, isem):
        pltpu.make_async_copy(sidx_hbm.at[wid, n], sbuf, isem).wait()
        pltpu.make_async_copy(didx_hbm.at[wid, n], dbuf, isem).wait()

    for i in range(CH // 16):
        ones_v[pl.ds(i * 16, 16)] = jnp.full((16,), 1.0, jnp.float32)
        zcnt_v[pl.ds(i * 16, 16)] = jnp.zeros((16,), jnp.float32)

    def zfill(t, carry):
        zrow_v[t, pl.ds(0, 16)] = jnp.zeros((16,), jnp.float32)
        zrow_v[t, pl.ds(16, 16)] = jnp.zeros((16,), jnp.float32)
        zrow_v[t, pl.ds(32, 16)] = jnp.zeros((16,), jnp.float32)
        zrow_v[t, pl.ds(48, 16)] = jnp.zeros((16,), jnp.float32)
        return carry

    lax.fori_loop(0, ZR, zfill, 0)

    def zero_acc(with_counts):
        for q in range(RPW // ZR):
            pltpu.sync_copy(zrow_v, acc_s.at[pl.ds(sid * RPW + q * ZR, ZR)])
        if with_counts:
            for q in range(RPW // CH):
                pltpu.sync_copy(zcnt_v, cnt_s.at[pl.ds(sid * RPW + q * CH, CH)])

    def gat_start(sbuf, j, buf, sem):
        pltpu.async_copy(xs_s.at[sbuf.at[j]], buf, sem)

    def gat_wait(sbuf, j, buf, sem):
        pltpu.make_async_copy(xs_s.at[sbuf.at[j]], buf, sem).wait()

    def scat_start(dbuf, j, buf, sem, with_counts):
        pltpu.async_copy(buf, acc_s.at[dbuf.at[j]], sem, add=True)
        if with_counts:
            pltpu.async_copy(ones_v, cnt_s.at[dbuf.at[j]], csem, add=True)

    def scat_wait(dbuf, j, buf, sem):
        pltpu.make_async_copy(buf, acc_s.at[dbuf.at[j]], sem).wait()

    def process_block(n, sbuf, dbuf, isem, with_counts):
        idx_wait(n, sbuf, dbuf, isem)
        gat_start(sbuf, 0, rows[0], gsem[0])

        def quad(k, carry):
            for m in range(4):
                i = 4 * k + m

                @pl.when(i >= 3)
                def _(i=i, m=m):
                    scat_wait(dbuf, i - 3, rows[(m + 1) % 4], ssem[(m + 1) % 4])

                @pl.when(i + 1 < IDXB)
                def _(i=i, m=m):
                    gat_start(sbuf, i + 1, rows[(m + 1) % 4], gsem[(m + 1) % 4])

                gat_wait(sbuf, i, rows[m], gsem[m])
                scat_start(dbuf, i, rows[m], ssem[m], with_counts)
            return carry

        lax.fori_loop(0, IDXB // 4, quad, 0)

        for j in range(IDXB - 3, IDXB):
            scat_wait(dbuf, j, rows[j % 4], ssem[j % 4])
        if with_counts:
            def cdrain(_, carry):
                pltpu.make_async_copy(ones_v, cnt_s.at[dbuf.at[0]], csem).wait()
                return carry
            lax.fori_loop(0, IDXB, cdrain, 0)

        if n + 2 < NIB:
            idx_start(n + 2, sbuf, dbuf, isem)

    XL = N_NODES - (NS - 1) * XPW

    def stage_fire(h):
        @pl.when(sid < NS - 1)
        def _():
            pltpu.async_copy(x_hbm.at[pl.ds(sid * XPW, XPW), pl.ds(h * DH, DH)],
                             xs_s.at[pl.ds(sid * XPW, XPW)], g0)

        @pl.when(sid == NS - 1)
        def _():
            pltpu.async_copy(
                x_hbm.at[pl.ds((NS - 1) * XPW, XL), pl.ds(h * DH, DH)],
                xs_s.at[pl.ds((NS - 1) * XPW, XL)], g0)

    def stage_wait(h):
        @pl.when(sid < NS - 1)
        def _():
            pltpu.make_async_copy(
                x_hbm.at[pl.ds(sid * XPW, XPW), pl.ds(h * DH, DH)],
                xs_s.at[pl.ds(sid * XPW, XPW)], g0).wait()

        @pl.when(sid == NS - 1)
        def _():
            pltpu.make_async_copy(
                x_hbm.at[pl.ds((NS - 1) * XPW, XL), pl.ds(h * DH, DH)],
                xs_s.at[pl.ds((NS - 1) * XPW, XL)], g0).wait()

    def run_blocks(with_counts):
        for n in range(NIB):
            if n % 2 == 0:
                process_block(n, sA_v, dA_v, isemA, with_counts)
            else:
                process_block(n, sB_v, dB_v, isemB, with_counts)

    def copyout(h, with_counts):
        @pl.when(sid < NS - 1)
        def _():
            pltpu.sync_copy(acc_s.at[pl.ds(sid * XPW, XPW)],
                            part_hbm.at[h, cid, pl.ds(sid * XPW, XPW)])
            if with_counts:
                pltpu.sync_copy(cnt_s.at[pl.ds(sid * XPW, XPW)],
                                cnt_hbm.at[cid, pl.ds(sid * XPW, XPW)])

        @pl.when(sid == NS - 1)
        def _():
            pltpu.sync_copy(acc_s.at[pl.ds((NS - 1) * XPW, XL)],
                            part_hbm.at[h, cid, pl.ds((NS - 1) * XPW, XL)])
            if with_counts:
                pltpu.sync_copy(cnt_s.at[pl.ds((NS - 1) * XPW, XL)],
                                cnt_hbm.at[cid, pl.ds((NS - 1) * XPW, XL)])

    stage_fire(0)
    idx_start(0, sA_v, dA_v, isemA)
    idx_start(1, sB_v, dB_v, isemB)
    zero_acc(True)
    stage_wait(0)
    plsc.subcore_barrier()
    run_blocks(True)
    plsc.subcore_barrier()

    stage_fire(1)
    idx_start(0, sA_v, dA_v, isemA)
    idx_start(1, sB_v, dB_v, isemB)
    copyout(0, True)
    zero_acc(False)
    stage_wait(1)
    plsc.subcore_barrier()
    run_blocks(False)
    plsc.subcore_barrier()
    copyout(1, False)


_sc_agg = pl.kernel(
    _sc_body,
    out_type=[
        jax.ShapeDtypeStruct((2, NC, N_NODES, DH), jnp.float32),
        jax.ShapeDtypeStruct((NC, N_NODES), jnp.float32),
    ],
    mesh=plsc.VectorSubcoreMesh(core_axis_name="c", subcore_axis_name="s"),
    compiler_params=pltpu.CompilerParams(use_tc_tiling_on_sc=False),
    scratch_types=[
        pltpu.VMEM((IDXB, CH), jnp.int32),
        pltpu.VMEM((IDXB, CH), jnp.int32),
        pltpu.VMEM((IDXB, CH), jnp.int32),
        pltpu.VMEM((IDXB, CH), jnp.int32),
        pltpu.VMEM((CH, DH), jnp.float32),
        pltpu.VMEM((CH, DH), jnp.float32),
        pltpu.VMEM((CH, DH), jnp.float32),
        pltpu.VMEM((CH, DH), jnp.float32),
        pltpu.VMEM((CH,), jnp.float32),
        pltpu.VMEM((ZR, DH), jnp.float32),
        pltpu.VMEM((CH,), jnp.float32),
        pltpu.VMEM_SHARED((N_NODES, DH), jnp.float32),
        pltpu.VMEM_SHARED((R_PAD, DH), jnp.float32),
        pltpu.VMEM_SHARED((R_PAD,), jnp.float32),
        pltpu.SemaphoreType.DMA,
        pltpu.SemaphoreType.DMA,
        pltpu.SemaphoreType.DMA,
        pltpu.SemaphoreType.DMA,
        pltpu.SemaphoreType.DMA,
        pltpu.SemaphoreType.DMA,
        pltpu.SemaphoreType.DMA,
        pltpu.SemaphoreType.DMA,
        pltpu.SemaphoreType.DMA,
        pltpu.SemaphoreType.DMA,
        pltpu.SemaphoreType.DMA,
    ],
)

BM = 1000
NB = N_NODES // (2 * BM)
D2 = 2 * D


def _tc_body(part_ref, inv_ref, has_ref, m0_ref, m1_ref, b_ref, g_ref, bt_ref,
             out_ref, pre_vmem, stat_vmem):
    i = pl.program_id(0)

    @pl.when(i < NB)
    def _():
        sc0 = (part_ref[0, 0] + part_ref[0, 1]) * inv_ref[...]
        sc1 = (part_ref[1, 0] + part_ref[1, 1]) * inv_ref[...]
        pre = lax.dot_general(sc0, m0_ref[...], (((1,), (0,)), ((), ())),
                              preferred_element_type=jnp.float32)
        pre += lax.dot_general(sc1, m1_ref[...], (((1,), (0,)), ((), ())),
                               preferred_element_type=jnp.float32)
        hb = has_ref[...]
        pre += jnp.concatenate([hb[:, 0:1] * b_ref[...],
                                hb[:, 1:2] * b_ref[...]], axis=1)
        pre_vmem[pl.ds(i * BM, BM), :] = pre

        @pl.when(i == 0)
        def _():
            stat_vmem[...] = jnp.zeros_like(stat_vmem)

        stat_vmem[0:1] += jnp.sum(pre, axis=0, keepdims=True)
        stat_vmem[1:2] += jnp.sum(pre * pre, axis=0, keepdims=True)

    @pl.when(i >= NB)
    def _():
        inv_n = 1.0 / N_NODES
        mu = (stat_vmem[0:1, 0:D] + stat_vmem[0:1, D:D2]) * inv_n
        ex2 = (stat_vmem[1:2, 0:D] + stat_vmem[1:2, D:D2]) * inv_n
        var = ex2 - mu * mu
        scale = g_ref[...] * lax.rsqrt(var + BN_EPS)
        mu2 = jnp.concatenate([mu, mu], axis=1)
        scale2 = jnp.concatenate([scale, scale], axis=1)
        bt2 = jnp.concatenate([bt_ref[...], bt_ref[...]], axis=1)
        pre = pre_vmem[pl.ds((i - NB) * BM, BM), :]
        out_ref[...] = jnp.maximum((pre - mu2) * scale2 + bt2, 0.0)


_tc_fused = pl.pallas_call(
    _tc_body,
    grid=(2 * NB,),
    in_specs=[
        pl.BlockSpec((2, NC, BM, D), lambda i: (0, 0, jnp.minimum(i, NB - 1), 0)),
        pl.BlockSpec((BM, D), lambda i: (jnp.minimum(i, NB - 1), 0)),
        pl.BlockSpec((BM, 2), lambda i: (jnp.minimum(i, NB - 1), 0)),
        pl.BlockSpec((D, D2), lambda i: (0, 0)),
        pl.BlockSpec((D, D2), lambda i: (0, 0)),
        pl.BlockSpec((1, D), lambda i: (0, 0)),
        pl.BlockSpec((1, D), lambda i: (0, 0)),
        pl.BlockSpec((1, D), lambda i: (0, 0)),
    ],
    out_specs=pl.BlockSpec((BM, D2), lambda i: (jnp.maximum(i - NB, 0), 0)),
    out_shape=jax.ShapeDtypeStruct((N_NODES // 2, D2), jnp.float32),
    scratch_shapes=[
        pltpu.VMEM((N_NODES // 2, D2), jnp.float32),
        pltpu.VMEM((2, D2), jnp.float32),
    ],
)


def kernel(x, edge_index, W, b, gamma, beta):
    ei = edge_index.astype(jnp.int32)
    pad = E_PAD - N_EDGES
    src = jnp.concatenate([ei[0], jnp.zeros((pad,), jnp.int32)])
    dst = jnp.concatenate([ei[1], jnp.full((pad,), R_PAD - 1, jnp.int32)])
    src = src.reshape(NW, NIB, IDXB, CH)
    dst = dst.reshape(NW, NIB, IDXB, CH)
    part, cnt = _sc_agg(x, src, dst)

    NP = N_NODES // 2
    part2 = part.reshape(2, NC, NP, D)
    c = cnt[0] + cnt[1]
    inv = 1.0 / jnp.clip(c, 1.0, None)
    inv2 = jnp.broadcast_to(inv.reshape(NP, 2, 1),
                            (NP, 2, DH)).reshape(NP, D)
    has2 = (c > 0).astype(jnp.float32).reshape(NP, 2)
    w0t = W[:, :DH].T
    w1t = W[:, DH:].T
    z = jnp.zeros((DH, D), jnp.float32)
    m0 = jnp.concatenate([jnp.concatenate([w0t, z], 1),
                          jnp.concatenate([z, w0t], 1)], 0)
    m1 = jnp.concatenate([jnp.concatenate([w1t, z], 1),
                          jnp.concatenate([z, w1t], 1)], 0)

    out2 = _tc_fused(part2, inv2, has2, m0, m1, b.reshape(1, D),
                     gamma.reshape(1, D), beta.reshape(1, D))
    return out2.reshape(N_NODES, D)


# speedup vs baseline: 1.5277x; 1.0321x over previous
import functools

import jax
import jax.numpy as jnp
from jax import lax
from jax.experimental import pallas as pl
from jax.experimental.pallas import tpu as pltpu
from jax.experimental.pallas import tpu_sc as plsc

N_NODES = 10000
N_EDGES = 320000
D = 128
DH = D // 2
BN_EPS = 1e-5

NC = 2
NS = 16
NW = NC * NS
CH = 128
STEPS = 80
IDXB = 16
NIB = STEPS // IDXB
E_PER_W = STEPS * CH
E_PAD = NW * E_PER_W
R_PAD = 10240
RPW = R_PAD // NS
XPW = 632
ZR = 64


def _sc_body(x_hbm, sidx_hbm, didx_hbm,
             part_hbm, cnt_hbm,
             sA_v, dA_v, sB_v, dB_v, rows0_v, rows1_v, rows2_v, rows3_v,
             ones_v, zrow_v, zcnt_v, xs_s, acc_s, cnt_s,
             g0, g1, g2, g3, s0, s1, s2, s3, csem, isemA, isemB):
    cid = lax.axis_index("c")
    sid = lax.axis_index("s")
    wid = cid * NS + sid
    rows = (rows0_v, rows1_v, rows2_v, rows3_v)
    gsem = (g0, g1, g2, g3)
    ssem = (s0, s1, s2, s3)

    def idx_start(n, sbuf, dbuf, isem):
        pltpu.async_copy(sidx_hbm.at[wid, n], sbuf, isem)
        pltpu.async_copy(didx_hbm.at[wid, n], dbuf, isem)

    def idx_wait(n, sbuf, dbuf, isem):
        pltpu.make_async_copy(sidx_hbm.at[wid, n], sbuf, isem).wait()
        pltpu.make_async_copy(didx_hbm.at[wid, n], dbuf, isem).wait()

    for i in range(CH // 16):
        ones_v[pl.ds(i * 16, 16)] = jnp.full((16,), 1.0, jnp.float32)
        zcnt_v[pl.ds(i * 16, 16)] = jnp.zeros((16,), jnp.float32)

    def zfill(t, carry):
        zrow_v[t, pl.ds(0, 16)] = jnp.zeros((16,), jnp.float32)
        zrow_v[t, pl.ds(16, 16)] = jnp.zeros((16,), jnp.float32)
        zrow_v[t, pl.ds(32, 16)] = jnp.zeros((16,), jnp.float32)
        zrow_v[t, pl.ds(48, 16)] = jnp.zeros((16,), jnp.float32)
        return carry

    lax.fori_loop(0, ZR, zfill, 0)

    def zero_acc(with_counts):
        for q in range(RPW // ZR):
            pltpu.sync_copy(zrow_v, acc_s.at[pl.ds(sid * RPW + q * ZR, ZR)])
        if with_counts:
            for q in range(RPW // CH):
                pltpu.sync_copy(zcnt_v, cnt_s.at[pl.ds(sid * RPW + q * CH, CH)])

    def gat_start(sbuf, j, buf, sem):
        pltpu.async_copy(xs_s.at[sbuf.at[j]], buf, sem)

    def gat_wait(sbuf, j, buf, sem):
        pltpu.make_async_copy(xs_s.at[sbuf.at[j]], buf, sem).wait()

    def scat_start(dbuf, j, buf, sem, with_counts):
        pltpu.async_copy(buf, acc_s.at[dbuf.at[j]], sem, add=True)
        if with_counts:
            pltpu.async_copy(ones_v, cnt_s.at[dbuf.at[j]], csem, add=True)

    def scat_wait(dbuf, j, buf, sem):
        pltpu.make_async_copy(buf, acc_s.at[dbuf.at[j]], sem).wait()

    def process_block(n, sbuf, dbuf, isem, with_counts):
        idx_wait(n, sbuf, dbuf, isem)
        gat_start(sbuf, 0, rows[0], gsem[0])

        def quad(k, carry):
            for m in range(4):
                i = 4 * k + m

                @pl.when(i >= 3)
                def _(i=i, m=m):
                    scat_wait(dbuf, i - 3, rows[(m + 1) % 4], ssem[(m + 1) % 4])

                @pl.when(i + 1 < IDXB)
                def _(i=i, m=m):
                    gat_start(sbuf, i + 1, rows[(m + 1) % 4], gsem[(m + 1) % 4])

                gat_wait(sbuf, i, rows[m], gsem[m])
                scat_start(dbuf, i, rows[m], ssem[m], with_counts)
            return carry

        lax.fori_loop(0, IDXB // 4, quad, 0)

        for j in range(IDXB - 3, IDXB):
            scat_wait(dbuf, j, rows[j % 4], ssem[j % 4])
        if with_counts:
            def cdrain(_, carry):
                pltpu.make_async_copy(ones_v, cnt_s.at[dbuf.at[0]], csem).wait()
                return carry
            lax.fori_loop(0, IDXB, cdrain, 0)

        if n + 2 < NIB:
            idx_start(n + 2, sbuf, dbuf, isem)

    XL = N_NODES - (NS - 1) * XPW

    def stage_fire(h):
        @pl.when(sid < NS - 1)
        def _():
            pltpu.async_copy(x_hbm.at[pl.ds(sid * XPW, XPW), pl.ds(h * DH, DH)],
                             xs_s.at[pl.ds(sid * XPW, XPW)], g0)

        @pl.when(sid == NS - 1)
        def _():
            pltpu.async_copy(
                x_hbm.at[pl.ds((NS - 1) * XPW, XL), pl.ds(h * DH, DH)],
                xs_s.at[pl.ds((NS - 1) * XPW, XL)], g0)

    def stage_wait(h):
        @pl.when(sid < NS - 1)
        def _():
            pltpu.make_async_copy(
                x_hbm.at[pl.ds(sid * XPW, XPW), pl.ds(h * DH, DH)],
                xs_s.at[pl.ds(sid * XPW, XPW)], g0).wait()

        @pl.when(sid == NS - 1)
        def _():
            pltpu.make_async_copy(
                x_hbm.at[pl.ds((NS - 1) * XPW, XL), pl.ds(h * DH, DH)],
                xs_s.at[pl.ds((NS - 1) * XPW, XL)], g0).wait()

    def run_blocks(with_counts):
        for n in range(NIB):
            if n % 2 == 0:
                process_block(n, sA_v, dA_v, isemA, with_counts)
            else:
                process_block(n, sB_v, dB_v, isemB, with_counts)

    def copyout(h, with_counts):
        @pl.when(sid < NS - 1)
        def _():
            pltpu.sync_copy(acc_s.at[pl.ds(sid * XPW, XPW)],
                            part_hbm.at[h, cid, pl.ds(sid * XPW, XPW)])
            if with_counts:
                pltpu.sync_copy(cnt_s.at[pl.ds(sid * XPW, XPW)],
                                cnt_hbm.at[cid, pl.ds(sid * XPW, XPW)])

        @pl.when(sid == NS - 1)
        def _():
            pltpu.sync_copy(acc_s.at[pl.ds((NS - 1) * XPW, XL)],
                            part_hbm.at[h, cid, pl.ds((NS - 1) * XPW, XL)])
            if with_counts:
                pltpu.sync_copy(cnt_s.at[pl.ds((NS - 1) * XPW, XL)],
                                cnt_hbm.at[cid, pl.ds((NS - 1) * XPW, XL)])

    stage_fire(0)
    idx_start(0, sA_v, dA_v, isemA)
    idx_start(1, sB_v, dB_v, isemB)
    zero_acc(True)
    stage_wait(0)
    plsc.subcore_barrier()
    run_blocks(True)
    plsc.subcore_barrier()

    stage_fire(1)
    idx_start(0, sA_v, dA_v, isemA)
    idx_start(1, sB_v, dB_v, isemB)
    copyout(0, True)
    zero_acc(False)
    stage_wait(1)
    plsc.subcore_barrier()
    run_blocks(False)
    plsc.subcore_barrier()
    copyout(1, False)


_sc_agg = pl.kernel(
    _sc_body,
    out_type=[
        jax.ShapeDtypeStruct((2, NC, N_NODES, DH), jnp.float32),
        jax.ShapeDtypeStruct((NC, N_NODES), jnp.float32),
    ],
    mesh=plsc.VectorSubcoreMesh(core_axis_name="c", subcore_axis_name="s"),
    compiler_params=pltpu.CompilerParams(use_tc_tiling_on_sc=False),
    scratch_types=[
        pltpu.VMEM((IDXB, CH), jnp.int32),
        pltpu.VMEM((IDXB, CH), jnp.int32),
        pltpu.VMEM((IDXB, CH), jnp.int32),
        pltpu.VMEM((IDXB, CH), jnp.int32),
        pltpu.VMEM((CH, DH), jnp.float32),
        pltpu.VMEM((CH, DH), jnp.float32),
        pltpu.VMEM((CH, DH), jnp.float32),
        pltpu.VMEM((CH, DH), jnp.float32),
        pltpu.VMEM((CH,), jnp.float32),
        pltpu.VMEM((ZR, DH), jnp.float32),
        pltpu.VMEM((CH,), jnp.float32),
        pltpu.VMEM_SHARED((N_NODES, DH), jnp.float32),
        pltpu.VMEM_SHARED((R_PAD, DH), jnp.float32),
        pltpu.VMEM_SHARED((R_PAD,), jnp.float32),
        pltpu.SemaphoreType.DMA,
        pltpu.SemaphoreType.DMA,
        pltpu.SemaphoreType.DMA,
        pltpu.SemaphoreType.DMA,
        pltpu.SemaphoreType.DMA,
        pltpu.SemaphoreType.DMA,
        pltpu.SemaphoreType.DMA,
        pltpu.SemaphoreType.DMA,
        pltpu.SemaphoreType.DMA,
        pltpu.SemaphoreType.DMA,
        pltpu.SemaphoreType.DMA,
    ],
)

BM = 1000
NB = N_NODES // (2 * BM)
D2 = 2 * D


def _tc_body(part_ref, inv_ref, has_ref, m0_ref, m1_ref, b_ref, g_ref, bt_ref,
             out_ref, pre_vmem, stat_vmem):
    i = pl.program_id(0)

    @pl.when(i < NB)
    def _():
        sc0 = (part_ref[0, 0] + part_ref[0, 1]) * inv_ref[...]
        sc1 = (part_ref[1, 0] + part_ref[1, 1]) * inv_ref[...]
        pre = lax.dot_general(sc0, m0_ref[...], (((1,), (0,)), ((), ())),
                              preferred_element_type=jnp.float32)
        pre += lax.dot_general(sc1, m1_ref[...], (((1,), (0,)), ((), ())),
                               preferred_element_type=jnp.float32)
        hb = has_ref[...]
        pre += jnp.concatenate([hb[:, 0:1] * b_ref[...],
                                hb[:, 1:2] * b_ref[...]], axis=1)
        pre_vmem[pl.ds(i * BM, BM), :] = pre

        @pl.when(i == 0)
        def _():
            stat_vmem[...] = jnp.zeros_like(stat_vmem)

        stat_vmem[0:1] += jnp.sum(pre, axis=0, keepdims=True)
        stat_vmem[1:2] += jnp.sum(pre * pre, axis=0, keepdims=True)

    @pl.when(i >= NB)
    def _():
        inv_n = 1.0 / N_NODES
        mu = (stat_vmem[0:1, 0:D] + stat_vmem[0:1, D:D2]) * inv_n
        ex2 = (stat_vmem[1:2, 0:D] + stat_vmem[1:2, D:D2]) * inv_n
        var = ex2 - mu * mu
        scale = g_ref[...] * lax.rsqrt(var + BN_EPS)
        mu2 = jnp.concatenate([mu, mu], axis=1)
        scale2 = jnp.concatenate([scale, scale], axis=1)
        bt2 = jnp.concatenate([bt_ref[...], bt_ref[...]], axis=1)
        pre = pre_vmem[pl.ds((i - NB) * BM, BM), :]
        res = jnp.maximum((pre - mu2) * scale2 + bt2, 0.0)
        out_ref[...] = res.reshape(2 * BM, D)


_tc_fused = pl.pallas_call(
    _tc_body,
    grid=(2 * NB,),
    in_specs=[
        pl.BlockSpec((2, NC, BM, D), lambda i: (0, 0, jnp.minimum(i, NB - 1), 0)),
        pl.BlockSpec((BM, D), lambda i: (jnp.minimum(i, NB - 1), 0)),
        pl.BlockSpec((BM, 2), lambda i: (jnp.minimum(i, NB - 1), 0)),
        pl.BlockSpec((D, D2), lambda i: (0, 0)),
        pl.BlockSpec((D, D2), lambda i: (0, 0)),
        pl.BlockSpec((1, D), lambda i: (0, 0)),
        pl.BlockSpec((1, D), lambda i: (0, 0)),
        pl.BlockSpec((1, D), lambda i: (0, 0)),
    ],
    out_specs=pl.BlockSpec((2 * BM, D), lambda i: (jnp.maximum(i - NB, 0), 0)),
    out_shape=jax.ShapeDtypeStruct((N_NODES, D), jnp.float32),
    scratch_shapes=[
        pltpu.VMEM((N_NODES // 2, D2), jnp.float32),
        pltpu.VMEM((2, D2), jnp.float32),
    ],
)


def kernel(x, edge_index, W, b, gamma, beta):
    ei = edge_index.astype(jnp.int32)
    pad = E_PAD - N_EDGES
    src = jnp.concatenate([ei[0], jnp.zeros((pad,), jnp.int32)])
    dst = jnp.concatenate([ei[1], jnp.full((pad,), R_PAD - 1, jnp.int32)])
    src = src.reshape(NW, NIB, IDXB, CH)
    dst = dst.reshape(NW, NIB, IDXB, CH)
    part, cnt = _sc_agg(x, src, dst)

    NP = N_NODES // 2
    part2 = part.reshape(2, NC, NP, D)
    c = cnt[0] + cnt[1]
    inv = 1.0 / jnp.clip(c, 1.0, None)
    inv2 = jnp.broadcast_to(inv.reshape(NP, 2, 1),
                            (NP, 2, DH)).reshape(NP, D)
    has2 = (c > 0).astype(jnp.float32).reshape(NP, 2)
    w0t = W[:, :DH].T
    w1t = W[:, DH:].T
    z = jnp.zeros((DH, D), jnp.float32)
    m0 = jnp.concatenate([jnp.concatenate([w0t, z], 1),
                          jnp.concatenate([z, w0t], 1)], 0)
    m1 = jnp.concatenate([jnp.concatenate([w1t, z], 1),
                          jnp.concatenate([z, w1t], 1)], 0)

    return _tc_fused(part2, inv2, has2, m0, m1, b.reshape(1, D),
                     gamma.reshape(1, D), beta.reshape(1, D))


# speedup vs baseline: 1.5280x; 1.0002x over previous
import jax
import jax.numpy as jnp
from jax import lax
from jax.experimental import pallas as pl
from jax.experimental.pallas import tpu as pltpu
from jax.experimental.pallas import tpu_sc as plsc

N_NODES = 10000
N_EDGES = 320000
D = 128
DH = D // 2
BN_EPS = 1e-5

NC = 2
NS = 16
NW = NC * NS
CH = 128
STEPS = 80
IDXB = 16
NIB = STEPS // IDXB
E_PER_W = STEPS * CH
E_PAD = NW * E_PER_W
R_PAD = 10240
RPW = R_PAD // NS
XPW = 632
ZR = 64


def _sc_body(x_hbm, sidx_hbm, didx_hbm,
             part_hbm, cnt_hbm,
             sA_v, dA_v, sB_v, dB_v, rows0_v, rows1_v, rows2_v, rows3_v,
             ones_v, zrow_v, zcnt_v, xs_s, acc_s, cnt_s,
             g0, g1, g2, g3, s0, s1, s2, s3, csem, isemA, isemB):
    cid = lax.axis_index("c")
    sid = lax.axis_index("s")
    wid = cid * NS + sid
    rows = (rows0_v, rows1_v, rows2_v, rows3_v)
    gsem = (g0, g1, g2, g3)
    ssem = (s0, s1, s2, s3)

    def idx_start(n, sbuf, dbuf, isem):
        pltpu.async_copy(sidx_hbm.at[wid, n], sbuf, isem)
        pltpu.async_copy(didx_hbm.at[wid, n], dbuf, isem)

    def idx_wait(n, sbuf, dbuf, isem):
        pltpu.make_async_copy(sidx_hbm.at[wid, n], sbuf, isem).wait()
        pltpu.make_async_copy(didx_hbm.at[wid, n], dbuf, isem).wait()

    for i in range(CH // 16):
        ones_v[pl.ds(i * 16, 16)] = jnp.full((16,), 1.0, jnp.float32)
        zcnt_v[pl.ds(i * 16, 16)] = jnp.zeros((16,), jnp.float32)

    def zfill(t, carry):
        zrow_v[t, pl.ds(0, 16)] = jnp.zeros((16,), jnp.float32)
        zrow_v[t, pl.ds(16, 16)] = jnp.zeros((16,), jnp.float32)
        zrow_v[t, pl.ds(32, 16)] = jnp.zeros((16,), jnp.float32)
        zrow_v[t, pl.ds(48, 16)] = jnp.zeros((16,), jnp.float32)
        return carry

    lax.fori_loop(0, ZR, zfill, 0)

    def zero_acc(with_counts):
        for q in range(RPW // ZR):
            pltpu.sync_copy(zrow_v, acc_s.at[pl.ds(sid * RPW + q * ZR, ZR)])
        if with_counts:
            for q in range(RPW // CH):
                pltpu.sync_copy(zcnt_v, cnt_s.at[pl.ds(sid * RPW + q * CH, CH)])

    def gat_start(sbuf, j, buf, sem):
        pltpu.async_copy(xs_s.at[sbuf.at[j]], buf, sem)

    def gat_wait(sbuf, j, buf, sem):
        pltpu.make_async_copy(xs_s.at[sbuf.at[j]], buf, sem).wait()

    def scat_start(dbuf, j, buf, sem, with_counts):
        pltpu.async_copy(buf, acc_s.at[dbuf.at[j]], sem, add=True)
        if with_counts:
            pltpu.async_copy(ones_v, cnt_s.at[dbuf.at[j]], csem, add=True)

    def scat_wait(dbuf, j, buf, sem):
        pltpu.make_async_copy(buf, acc_s.at[dbuf.at[j]], sem).wait()

    def process_block(n, sbuf, dbuf, isem, with_counts):
        idx_wait(n, sbuf, dbuf, isem)
        gat_start(sbuf, 0, rows[0], gsem[0])

        def quad(k, carry):
            for m in range(4):
                i = 4 * k + m

                @pl.when(i >= 3)
                def _(i=i, m=m):
                    scat_wait(dbuf, i - 3, rows[(m + 1) % 4], ssem[(m + 1) % 4])

                @pl.when(i + 1 < IDXB)
                def _(i=i, m=m):
                    gat_start(sbuf, i + 1, rows[(m + 1) % 4], gsem[(m + 1) % 4])

                gat_wait(sbuf, i, rows[m], gsem[m])
                scat_start(dbuf, i, rows[m], ssem[m], with_counts)
            return carry

        lax.fori_loop(0, IDXB // 4, quad, 0)

        for j in range(IDXB - 3, IDXB):
            scat_wait(dbuf, j, rows[j % 4], ssem[j % 4])
        if with_counts:
            def cdrain(_, carry):
                pltpu.make_async_copy(ones_v, cnt_s.at[dbuf.at[0]], csem).wait()
                return carry
            lax.fori_loop(0, IDXB, cdrain, 0)

        if n + 2 < NIB:
            idx_start(n + 2, sbuf, dbuf, isem)

    XL = N_NODES - (NS - 1) * XPW

    def stage_fire(h):
        @pl.when(sid < NS - 1)
        def _():
            pltpu.async_copy(x_hbm.at[pl.ds(sid * XPW, XPW), pl.ds(h * DH, DH)],
                             xs_s.at[pl.ds(sid * XPW, XPW)], g0)

        @pl.when(sid == NS - 1)
        def _():
            pltpu.async_copy(
                x_hbm.at[pl.ds((NS - 1) * XPW, XL), pl.ds(h * DH, DH)],
                xs_s.at[pl.ds((NS - 1) * XPW, XL)], g0)

    def stage_wait(h):
        @pl.when(sid < NS - 1)
        def _():
            pltpu.make_async_copy(
                x_hbm.at[pl.ds(sid * XPW, XPW), pl.ds(h * DH, DH)],
                xs_s.at[pl.ds(sid * XPW, XPW)], g0).wait()

        @pl.when(sid == NS - 1)
        def _():
            pltpu.make_async_copy(
                x_hbm.at[pl.ds((NS - 1) * XPW, XL), pl.ds(h * DH, DH)],
                xs_s.at[pl.ds((NS - 1) * XPW, XL)], g0).wait()

    def run_blocks(with_counts):
        for n in range(NIB):
            if n % 2 == 0:
                process_block(n, sA_v, dA_v, isemA, with_counts)
            else:
                process_block(n, sB_v, dB_v, isemB, with_counts)

    def copyout(h, with_counts):
        @pl.when(sid < NS - 1)
        def _():
            pltpu.sync_copy(acc_s.at[pl.ds(sid * XPW, XPW)],
                            part_hbm.at[h, cid, pl.ds(sid * XPW, XPW)])
            if with_counts:
                pltpu.sync_copy(cnt_s.at[pl.ds(sid * XPW, XPW)],
                                cnt_hbm.at[cid, pl.ds(sid * XPW, XPW)])

        @pl.when(sid == NS - 1)
        def _():
            pltpu.sync_copy(acc_s.at[pl.ds((NS - 1) * XPW, XL)],
                            part_hbm.at[h, cid, pl.ds((NS - 1) * XPW, XL)])
            if with_counts:
                pltpu.sync_copy(cnt_s.at[pl.ds((NS - 1) * XPW, XL)],
                                cnt_hbm.at[cid, pl.ds((NS - 1) * XPW, XL)])

    stage_fire(0)
    idx_start(0, sA_v, dA_v, isemA)
    idx_start(1, sB_v, dB_v, isemB)
    zero_acc(True)
    stage_wait(0)
    plsc.subcore_barrier()
    run_blocks(True)
    plsc.subcore_barrier()

    stage_fire(1)
    idx_start(0, sA_v, dA_v, isemA)
    idx_start(1, sB_v, dB_v, isemB)
    copyout(0, True)
    zero_acc(False)
    stage_wait(1)
    plsc.subcore_barrier()
    run_blocks(False)
    plsc.subcore_barrier()
    copyout(1, False)


_sc_agg = pl.kernel(
    _sc_body,
    out_type=[
        jax.ShapeDtypeStruct((2, NC, N_NODES, DH), jnp.float32),
        jax.ShapeDtypeStruct((NC, N_NODES), jnp.float32),
    ],
    mesh=plsc.VectorSubcoreMesh(core_axis_name="c", subcore_axis_name="s"),
    compiler_params=pltpu.CompilerParams(use_tc_tiling_on_sc=False),
    scratch_types=[
        pltpu.VMEM((IDXB, CH), jnp.int32),
        pltpu.VMEM((IDXB, CH), jnp.int32),
        pltpu.VMEM((IDXB, CH), jnp.int32),
        pltpu.VMEM((IDXB, CH), jnp.int32),
        pltpu.VMEM((CH, DH), jnp.float32),
        pltpu.VMEM((CH, DH), jnp.float32),
        pltpu.VMEM((CH, DH), jnp.float32),
        pltpu.VMEM((CH, DH), jnp.float32),
        pltpu.VMEM((CH,), jnp.float32),
        pltpu.VMEM((ZR, DH), jnp.float32),
        pltpu.VMEM((CH,), jnp.float32),
        pltpu.VMEM_SHARED((N_NODES, DH), jnp.float32),
        pltpu.VMEM_SHARED((R_PAD, DH), jnp.float32),
        pltpu.VMEM_SHARED((R_PAD,), jnp.float32),
        pltpu.SemaphoreType.DMA,
        pltpu.SemaphoreType.DMA,
        pltpu.SemaphoreType.DMA,
        pltpu.SemaphoreType.DMA,
        pltpu.SemaphoreType.DMA,
        pltpu.SemaphoreType.DMA,
        pltpu.SemaphoreType.DMA,
        pltpu.SemaphoreType.DMA,
        pltpu.SemaphoreType.DMA,
        pltpu.SemaphoreType.DMA,
        pltpu.SemaphoreType.DMA,
    ],
)

BM = 1000
NB = N_NODES // (2 * BM)
D2 = 2 * D


def _tc_body(part_ref, inv_ref, has_ref, m0_ref, m1_ref, b_ref, g_ref, bt_ref,
             out_ref, pre_vmem, stat_vmem):
    i = pl.program_id(0)

    @pl.when(i < NB)
    def _():
        sc0 = (part_ref[0, 0] + part_ref[0, 1]) * inv_ref[...]
        sc1 = (part_ref[1, 0] + part_ref[1, 1]) * inv_ref[...]
        pre = lax.dot_general(sc0, m0_ref[...], (((1,), (0,)), ((), ())),
                              preferred_element_type=jnp.float32)
        pre += lax.dot_general(sc1, m1_ref[...], (((1,), (0,)), ((), ())),
                               preferred_element_type=jnp.float32)
        hb = has_ref[...]
        pre += jnp.concatenate([hb[:, 0:1] * b_ref[...],
                                hb[:, 1:2] * b_ref[...]], axis=1)
        pre_vmem[pl.ds(i * BM, BM), :] = pre

        @pl.when(i == 0)
        def _():
            stat_vmem[...] = jnp.zeros_like(stat_vmem)

        stat_vmem[0:1] += jnp.sum(pre, axis=0, keepdims=True)
        stat_vmem[1:2] += jnp.sum(pre * pre, axis=0, keepdims=True)

    @pl.when(i >= NB)
    def _():
        inv_n = 1.0 / N_NODES
        mu = (stat_vmem[0:1, 0:D] + stat_vmem[0:1, D:D2]) * inv_n
        ex2 = (stat_vmem[1:2, 0:D] + stat_vmem[1:2, D:D2]) * inv_n
        var = ex2 - mu * mu
        scale = g_ref[...] * lax.rsqrt(var + BN_EPS)
        mu2 = jnp.concatenate([mu, mu], axis=1)
        scale2 = jnp.concatenate([scale, scale], axis=1)
        bt2 = jnp.concatenate([bt_ref[...], bt_ref[...]], axis=1)
        pre = pre_vmem[pl.ds((i - NB) * BM, BM), :]
        res = jnp.maximum((pre - mu2) * scale2 + bt2, 0.0)
        out_ref[...] = res.reshape(2 * BM, D)


_tc_fused = pl.pallas_call(
    _tc_body,
    grid=(2 * NB,),
    in_specs=[
        pl.BlockSpec((2, NC, BM, D), lambda i: (0, 0, jnp.minimum(i, NB - 1), 0)),
        pl.BlockSpec((BM, D), lambda i: (jnp.minimum(i, NB - 1), 0)),
        pl.BlockSpec((BM, 2), lambda i: (jnp.minimum(i, NB - 1), 0)),
        pl.BlockSpec((D, D2), lambda i: (0, 0)),
        pl.BlockSpec((D, D2), lambda i: (0, 0)),
        pl.BlockSpec((1, D), lambda i: (0, 0)),
        pl.BlockSpec((1, D), lambda i: (0, 0)),
        pl.BlockSpec((1, D), lambda i: (0, 0)),
    ],
    out_specs=pl.BlockSpec((2 * BM, D), lambda i: (jnp.maximum(i - NB, 0), 0)),
    out_shape=jax.ShapeDtypeStruct((N_NODES, D), jnp.float32),
    scratch_shapes=[
        pltpu.VMEM((N_NODES // 2, D2), jnp.float32),
        pltpu.VMEM((2, D2), jnp.float32),
    ],
)


def kernel(x, edge_index, W, b, gamma, beta):
    ei = edge_index.astype(jnp.int32)
    pad = E_PAD - N_EDGES
    src = jnp.concatenate([ei[0], jnp.zeros((pad,), jnp.int32)])
    dst = jnp.concatenate([ei[1], jnp.full((pad,), R_PAD - 1, jnp.int32)])
    src = src.reshape(NW, NIB, IDXB, CH)
    dst = dst.reshape(NW, NIB, IDXB, CH)
    part, cnt = _sc_agg(x, src, dst)

    NP = N_NODES // 2
    part2 = part.reshape(2, NC, NP, D)
    c = cnt[0] + cnt[1]
    inv = 1.0 / jnp.clip(c, 1.0, None)
    inv2 = jnp.broadcast_to(inv.reshape(NP, 2, 1),
                            (NP, 2, DH)).reshape(NP, D)
    has2 = (c > 0).astype(jnp.float32).reshape(NP, 2)
    w0t = W[:, :DH].T
    w1t = W[:, DH:].T
    z = jnp.zeros((DH, D), jnp.float32)
    m0 = jnp.concatenate([jnp.concatenate([w0t, z], 1),
                          jnp.concatenate([z, w0t], 1)], 0)
    m1 = jnp.concatenate([jnp.concatenate([w1t, z], 1),
                          jnp.concatenate([z, w1t], 1)], 0)

    return _tc_fused(part2, inv2, has2, m0, m1, b.reshape(1, D),
                     gamma.reshape(1, D), beta.reshape(1, D))


# speedup vs baseline: 1.5625x; 1.0225x over previous
import jax
import jax.numpy as jnp
from jax import lax
from jax.experimental import pallas as pl
from jax.experimental.pallas import tpu as pltpu
from jax.experimental.pallas import tpu_sc as plsc

N_NODES = 10000
N_EDGES = 320000
D = 128
DH = D // 2
BN_EPS = 1e-5

NC = 2
NS = 16
NW = NC * NS
CH = 128
STEPS = 80
IDXB = 20
NIB = STEPS // IDXB
E_PER_W = STEPS * CH
E_PAD = NW * E_PER_W
R_PAD = 10240
RPW = R_PAD // NS
XPW = 632
ZR = 64


def _sc_body(x_hbm, sidx_hbm, didx_hbm,
             part_hbm, cnt_hbm,
             sA_v, dA_v, sB_v, dB_v, rows0_v, rows1_v, rows2_v, rows3_v,
             ones_v, zrow_v, zcnt_v, xs_s, acc_s, cnt_s,
             g0, g1, g2, g3, s0, s1, s2, s3, csem, isemA, isemB):
    cid = lax.axis_index("c")
    sid = lax.axis_index("s")
    wid = cid * NS + sid
    rows = (rows0_v, rows1_v, rows2_v, rows3_v)
    gsem = (g0, g1, g2, g3)
    ssem = (s0, s1, s2, s3)

    def idx_start(n, sbuf, dbuf, isem):
        pltpu.async_copy(sidx_hbm.at[wid, n], sbuf, isem)
        pltpu.async_copy(didx_hbm.at[wid, n], dbuf, isem)

    def idx_wait(n, sbuf, dbuf, isem):
        pltpu.make_async_copy(sidx_hbm.at[wid, n], sbuf, isem).wait()
        pltpu.make_async_copy(didx_hbm.at[wid, n], dbuf, isem).wait()

    for i in range(CH // 16):
        ones_v[pl.ds(i * 16, 16)] = jnp.full((16,), 1.0, jnp.float32)
        zcnt_v[pl.ds(i * 16, 16)] = jnp.zeros((16,), jnp.float32)

    def zfill(t, carry):
        zrow_v[t, pl.ds(0, 16)] = jnp.zeros((16,), jnp.float32)
        zrow_v[t, pl.ds(16, 16)] = jnp.zeros((16,), jnp.float32)
        zrow_v[t, pl.ds(32, 16)] = jnp.zeros((16,), jnp.float32)
        zrow_v[t, pl.ds(48, 16)] = jnp.zeros((16,), jnp.float32)
        return carry

    lax.fori_loop(0, ZR, zfill, 0)

    def zero_acc(with_counts):
        for q in range(RPW // ZR):
            pltpu.sync_copy(zrow_v, acc_s.at[pl.ds(sid * RPW + q * ZR, ZR)])
        if with_counts:
            for q in range(RPW // CH):
                pltpu.sync_copy(zcnt_v, cnt_s.at[pl.ds(sid * RPW + q * CH, CH)])

    def gat_start(sbuf, j, buf, sem):
        pltpu.async_copy(xs_s.at[sbuf.at[j]], buf, sem)

    def gat_wait(sbuf, j, buf, sem):
        pltpu.make_async_copy(xs_s.at[sbuf.at[j]], buf, sem).wait()

    def scat_start(dbuf, j, buf, sem, with_counts):
        pltpu.async_copy(buf, acc_s.at[dbuf.at[j]], sem, add=True)
        if with_counts:
            pltpu.async_copy(ones_v, cnt_s.at[dbuf.at[j]], csem, add=True)

    def scat_wait(dbuf, j, buf, sem):
        pltpu.make_async_copy(buf, acc_s.at[dbuf.at[j]], sem).wait()

    def process_block(n, sbuf, dbuf, isem, with_counts):
        idx_wait(n, sbuf, dbuf, isem)
        gat_start(sbuf, 0, rows[0], gsem[0])

        def quad(k, carry):
            for m in range(4):
                i = 4 * k + m

                @pl.when(i >= 3)
                def _(i=i, m=m):
                    scat_wait(dbuf, i - 3, rows[(m + 1) % 4], ssem[(m + 1) % 4])

                @pl.when(i + 1 < IDXB)
                def _(i=i, m=m):
                    gat_start(sbuf, i + 1, rows[(m + 1) % 4], gsem[(m + 1) % 4])

                gat_wait(sbuf, i, rows[m], gsem[m])
                scat_start(dbuf, i, rows[m], ssem[m], with_counts)
            return carry

        lax.fori_loop(0, IDXB // 4, quad, 0)

        for j in range(IDXB - 3, IDXB):
            scat_wait(dbuf, j, rows[j % 4], ssem[j % 4])
        if with_counts:
            def cdrain(_, carry):
                pltpu.make_async_copy(ones_v, cnt_s.at[dbuf.at[0]], csem).wait()
                return carry
            lax.fori_loop(0, IDXB, cdrain, 0)

        if n + 2 < NIB:
            idx_start(n + 2, sbuf, dbuf, isem)

    XL = N_NODES - (NS - 1) * XPW

    def stage_fire(h):
        @pl.when(sid < NS - 1)
        def _():
            pltpu.async_copy(x_hbm.at[pl.ds(sid * XPW, XPW), pl.ds(h * DH, DH)],
                             xs_s.at[pl.ds(sid * XPW, XPW)], g0)

        @pl.when(sid == NS - 1)
        def _():
            pltpu.async_copy(
                x_hbm.at[pl.ds((NS - 1) * XPW, XL), pl.ds(h * DH, DH)],
                xs_s.at[pl.ds((NS - 1) * XPW, XL)], g0)

    def stage_wait(h):
        @pl.when(sid < NS - 1)
        def _():
            pltpu.make_async_copy(
                x_hbm.at[pl.ds(sid * XPW, XPW), pl.ds(h * DH, DH)],
                xs_s.at[pl.ds(sid * XPW, XPW)], g0).wait()

        @pl.when(sid == NS - 1)
        def _():
            pltpu.make_async_copy(
                x_hbm.at[pl.ds((NS - 1) * XPW, XL), pl.ds(h * DH, DH)],
                xs_s.at[pl.ds((NS - 1) * XPW, XL)], g0).wait()

    def run_blocks(with_counts):
        for n in range(NIB):
            if n % 2 == 0:
                process_block(n, sA_v, dA_v, isemA, with_counts)
            else:
                process_block(n, sB_v, dB_v, isemB, with_counts)

    def copyout(h, with_counts):
        @pl.when(sid < NS - 1)
        def _():
            pltpu.sync_copy(acc_s.at[pl.ds(sid * XPW, XPW)],
                            part_hbm.at[h, cid, pl.ds(sid * XPW, XPW)])
            if with_counts:
                pltpu.sync_copy(cnt_s.at[pl.ds(sid * XPW, XPW)],
                                cnt_hbm.at[cid, pl.ds(sid * XPW, XPW)])

        @pl.when(sid == NS - 1)
        def _():
            pltpu.sync_copy(acc_s.at[pl.ds((NS - 1) * XPW, XL)],
                            part_hbm.at[h, cid, pl.ds((NS - 1) * XPW, XL)])
            if with_counts:
                pltpu.sync_copy(cnt_s.at[pl.ds((NS - 1) * XPW, XL)],
                                cnt_hbm.at[cid, pl.ds((NS - 1) * XPW, XL)])

    stage_fire(0)
    idx_start(0, sA_v, dA_v, isemA)
    idx_start(1, sB_v, dB_v, isemB)
    zero_acc(True)
    stage_wait(0)
    plsc.subcore_barrier()
    run_blocks(True)
    plsc.subcore_barrier()

    stage_fire(1)
    idx_start(0, sA_v, dA_v, isemA)
    idx_start(1, sB_v, dB_v, isemB)
    copyout(0, True)
    zero_acc(False)
    stage_wait(1)
    plsc.subcore_barrier()
    run_blocks(False)
    plsc.subcore_barrier()
    copyout(1, False)


_sc_agg = pl.kernel(
    _sc_body,
    out_type=[
        jax.ShapeDtypeStruct((2, NC, N_NODES, DH), jnp.float32),
        jax.ShapeDtypeStruct((NC, N_NODES), jnp.float32),
    ],
    mesh=plsc.VectorSubcoreMesh(core_axis_name="c", subcore_axis_name="s"),
    compiler_params=pltpu.CompilerParams(use_tc_tiling_on_sc=False),
    scratch_types=[
        pltpu.VMEM((IDXB, CH), jnp.int32),
        pltpu.VMEM((IDXB, CH), jnp.int32),
        pltpu.VMEM((IDXB, CH), jnp.int32),
        pltpu.VMEM((IDXB, CH), jnp.int32),
        pltpu.VMEM((CH, DH), jnp.float32),
        pltpu.VMEM((CH, DH), jnp.float32),
        pltpu.VMEM((CH, DH), jnp.float32),
        pltpu.VMEM((CH, DH), jnp.float32),
        pltpu.VMEM((CH,), jnp.float32),
        pltpu.VMEM((ZR, DH), jnp.float32),
        pltpu.VMEM((CH,), jnp.float32),
        pltpu.VMEM_SHARED((N_NODES, DH), jnp.float32),
        pltpu.VMEM_SHARED((R_PAD, DH), jnp.float32),
        pltpu.VMEM_SHARED((R_PAD,), jnp.float32),
        pltpu.SemaphoreType.DMA,
        pltpu.SemaphoreType.DMA,
        pltpu.SemaphoreType.DMA,
        pltpu.SemaphoreType.DMA,
        pltpu.SemaphoreType.DMA,
        pltpu.SemaphoreType.DMA,
        pltpu.SemaphoreType.DMA,
        pltpu.SemaphoreType.DMA,
        pltpu.SemaphoreType.DMA,
        pltpu.SemaphoreType.DMA,
        pltpu.SemaphoreType.DMA,
    ],
)

BM = 1000
NB = N_NODES // (2 * BM)
D2 = 2 * D


def _tc_body(part_ref, inv_ref, has_ref, m0_ref, m1_ref, b_ref, g_ref, bt_ref,
             out_ref, pre_vmem, stat_vmem):
    i = pl.program_id(0)

    @pl.when(i < NB)
    def _():
        sc0 = (part_ref[0, 0] + part_ref[0, 1]) * inv_ref[...]
        sc1 = (part_ref[1, 0] + part_ref[1, 1]) * inv_ref[...]
        pre = lax.dot_general(sc0, m0_ref[...], (((1,), (0,)), ((), ())),
                              preferred_element_type=jnp.float32)
        pre += lax.dot_general(sc1, m1_ref[...], (((1,), (0,)), ((), ())),
                               preferred_element_type=jnp.float32)
        hb = has_ref[...]
        pre += jnp.concatenate([hb[:, 0:1] * b_ref[...],
                                hb[:, 1:2] * b_ref[...]], axis=1)
        pre_vmem[pl.ds(i * BM, BM), :] = pre

        @pl.when(i == 0)
        def _():
            stat_vmem[...] = jnp.zeros_like(stat_vmem)

        stat_vmem[0:1] += jnp.sum(pre, axis=0, keepdims=True)
        stat_vmem[1:2] += jnp.sum(pre * pre, axis=0, keepdims=True)

    @pl.when(i >= NB)
    def _():
        inv_n = 1.0 / N_NODES
        mu = (stat_vmem[0:1, 0:D] + stat_vmem[0:1, D:D2]) * inv_n
        ex2 = (stat_vmem[1:2, 0:D] + stat_vmem[1:2, D:D2]) * inv_n
        var = ex2 - mu * mu
        scale = g_ref[...] * lax.rsqrt(var + BN_EPS)
        mu2 = jnp.concatenate([mu, mu], axis=1)
        scale2 = jnp.concatenate([scale, scale], axis=1)
        bt2 = jnp.concatenate([bt_ref[...], bt_ref[...]], axis=1)
        pre = pre_vmem[pl.ds((i - NB) * BM, BM), :]
        res = jnp.maximum((pre - mu2) * scale2 + bt2, 0.0)
        out_ref[...] = res.reshape(2 * BM, D)


_tc_fused = pl.pallas_call(
    _tc_body,
    grid=(2 * NB,),
    in_specs=[
        pl.BlockSpec((2, NC, BM, D), lambda i: (0, 0, jnp.minimum(i, NB - 1), 0)),
        pl.BlockSpec((BM, D), lambda i: (jnp.minimum(i, NB - 1), 0)),
        pl.BlockSpec((BM, 2), lambda i: (jnp.minimum(i, NB - 1), 0)),
        pl.BlockSpec((D, D2), lambda i: (0, 0)),
        pl.BlockSpec((D, D2), lambda i: (0, 0)),
        pl.BlockSpec((1, D), lambda i: (0, 0)),
        pl.BlockSpec((1, D), lambda i: (0, 0)),
        pl.BlockSpec((1, D), lambda i: (0, 0)),
    ],
    out_specs=pl.BlockSpec((2 * BM, D), lambda i: (jnp.maximum(i - NB, 0), 0)),
    out_shape=jax.ShapeDtypeStruct((N_NODES, D), jnp.float32),
    scratch_shapes=[
        pltpu.VMEM((N_NODES // 2, D2), jnp.float32),
        pltpu.VMEM((2, D2), jnp.float32),
    ],
)


def kernel(x, edge_index, W, b, gamma, beta):
    ei = edge_index.astype(jnp.int32)
    pad = E_PAD - N_EDGES
    src = jnp.concatenate([ei[0], jnp.zeros((pad,), jnp.int32)])
    dst = jnp.concatenate([ei[1], jnp.full((pad,), R_PAD - 1, jnp.int32)])
    src = src.reshape(NW, NIB, IDXB, CH)
    dst = dst.reshape(NW, NIB, IDXB, CH)
    part, cnt = _sc_agg(x, src, dst)

    NP = N_NODES // 2
    part2 = part.reshape(2, NC, NP, D)
    c = cnt[0] + cnt[1]
    inv = 1.0 / jnp.clip(c, 1.0, None)
    inv2 = jnp.broadcast_to(inv.reshape(NP, 2, 1),
                            (NP, 2, DH)).reshape(NP, D)
    has2 = (c > 0).astype(jnp.float32).reshape(NP, 2)
    w0t = W[:, :DH].T
    w1t = W[:, DH:].T
    z = jnp.zeros((DH, D), jnp.float32)
    m0 = jnp.concatenate([jnp.concatenate([w0t, z], 1),
                          jnp.concatenate([z, w0t], 1)], 0)
    m1 = jnp.concatenate([jnp.concatenate([w1t, z], 1),
                          jnp.concatenate([z, w1t], 1)], 0)

    return _tc_fused(part2, inv2, has2, m0, m1, b.reshape(1, D),
                     gamma.reshape(1, D), beta.reshape(1, D))
